# baseline (speedup 1.0000x reference)
import functools
import jax
import jax.numpy as jnp
from jax import lax
from jax.experimental import pallas as pl
from jax.experimental.pallas import tpu as pltpu
from jax.experimental.pallas import tpu_sc as plsc

B = 8
N = 262144
TPR = 16
QUART = N // TPR
LREG = QUART // 16
CK = 256
CHUNK = 16 * CK
NCH = LREG // CK
NB = 2048
SHIFTS = (0, 11, 22)
MASK = 0x7FF
TOPBIT = 0x7FFFFFFF
TPS = 4
QS = N // TPS
NCHS = QS // CHUNK


def _make_key(x, lab):
    signs = 2.0 * lab.astype(jnp.float32) - 1.0
    e = 1.0 - x * signs
    b = plsc.bitcast(e, jnp.int32)
    return jnp.where(b >= 0, b ^ TOPBIT, b)


def _key_to_err(key):
    b = jnp.where(key >= 0, key ^ TOPBIT, key)
    return plsc.bitcast(b, jnp.float32)


def _sc_body(x_hbm, lab_hbm, loss_hbm, ka, kb, la, lb,
             offh, dbase, ttot, bgrid, xbuf, kbuf, lbuf,
             istage, obuf, sk_sh, sl_sh, htot_sh, ones_sh, sem):
    cid = lax.axis_index("c")
    sid = lax.axis_index("s")
    wid = cid * 16 + sid
    t = sid
    ids = lax.iota(jnp.int32, 16)
    ones16 = jnp.ones((16,), jnp.int32)
    zeros16 = jnp.zeros((16,), jnp.int32)

    def clear_offh(_i, _c):
        for u in range(4):
            offh[pl.ds(_i * 64 + u * 16, 16)] = zeros16
        return 0

    def load_col(buf, j):
        return plsc.load_gather(buf, [ids * CK + jnp.broadcast_to(j, (16,))])

    for w in range(4):
        row = cid * 4 + w
        hq = row * N + t * QUART

        def stage_in(src, dst, c):
            descs = []
            for l in range(16):
                descs.append(pltpu.async_copy(
                    src.at[pl.ds(hq + l * LREG + c * CK, CK)],
                    dst.at[pl.ds(l * CK, CK)], sem))
            return descs

        for p in range(3):
            shift = SHIFTS[p]
            if p == 0:
                src_k, src_l = None, None
                out_k, out_l = ka, la
            elif p == 1:
                src_k, src_l = ka, la
                out_k, out_l = kb, lb
            else:
                src_k, src_l = kb, lb
                out_k, out_l = ka, la

            lax.fori_loop(0, NB * 16 // 64, clear_offh, 0)

            def a_chunk(c, _c):
                if p == 0:
                    d1 = stage_in(x_hbm, xbuf, c)
                    d2 = stage_in(lab_hbm, lbuf, c)
                    for d in d1 + d2:
                        d.wait()
                else:
                    for d in stage_in(src_k, kbuf, c):
                        d.wait()

                def a_col(j, _j):
                    if p == 0:
                        key = _make_key(load_col(xbuf, j), load_col(lbuf, j))
                    else:
                        key = load_col(kbuf, j)
                    dig = lax.shift_right_logical(key, shift) & MASK
                    plsc.addupdate_scatter(offh, [ids * NB + dig], ones16)
                    return 0

                lax.fori_loop(0, CK, a_col, 0)
                return 0

            lax.fori_loop(0, NCH, a_chunk, 0)

            def b1(i, _c):
                acc = zeros16
                for l in range(16):
                    v = offh[pl.ds(l * NB + i * 16, 16)]
                    offh[pl.ds(l * NB + i * 16, 16)] = acc
                    acc = acc + v
                ttot[pl.ds(i * 16, 16)] = acc
                return 0

            lax.fori_loop(0, NB // 16, b1, 0)

            pltpu.sync_copy(ttot, htot_sh.at[t])
            plsc.subcore_barrier()
            for tt in range(TPR):
                pltpu.sync_copy(htot_sh.at[tt], bgrid.at[pl.ds(tt * NB, NB)])

            def b4(i, base):
                g = [bgrid[pl.ds(tt * NB + i * 16, 16)] for tt in range(TPR)]
                tsum = g[0]
                for tt in range(1, TPR):
                    tsum = tsum + g[tt]
                cs = plsc.cumsum(tsum)
                pre = zeros16
                for tt in range(TPR - 1):
                    pre = pre + jnp.where(t > tt, g[tt], zeros16)
                dbase[pl.ds(i * 16, 16)] = (cs - tsum) + pre + base
                return base + jnp.sum(tsum)

            lax.fori_loop(0, NB // 16, b4, jnp.int32(0))

            def c_chunk(c, _c):
                if p == 0:
                    d1 = stage_in(x_hbm, xbuf, c)
                    d2 = stage_in(lab_hbm, lbuf, c)
                else:
                    d1 = stage_in(src_k, kbuf, c)
                    d2 = stage_in(src_l, lbuf, c)
                for d in d1 + d2:
                    d.wait()

                def c_col(j, _j):
                    jb = jnp.broadcast_to(j, (16,))
                    lane_slot = ids * CK + jb
                    if p == 0:
                        lg = load_col(lbuf, j)
                        key = _make_key(load_col(xbuf, j), lg)
                        plsc.store_scatter(kbuf, [lane_slot], key)
                    else:
                        key = load_col(kbuf, j)
                    dig = lax.shift_right_logical(key, shift) & MASK
                    p0 = plsc.load_gather(offh, [ids * NB + dig])
                    gb = plsc.load_gather(dbase, [dig])
                    plsc.store_scatter(offh, [ids * NB + dig], p0 + 1)
                    plsc.store_scatter(istage, [lane_slot], p0 + gb)
                    return 0

                lax.fori_loop(0, CK, c_col, 0)
                pltpu.sync_copy(kbuf, sk_sh.at[istage])
                pltpu.sync_copy(lbuf, sl_sh.at[istage])
                return 0

            lax.fori_loop(0, NCH, c_chunk, 0)
            plsc.subcore_barrier()
            pltpu.sync_copy(sk_sh.at[pl.ds(t * QUART, QUART)],
                            out_k.at[pl.ds(hq, QUART)])
            pltpu.sync_copy(sl_sh.at[pl.ds(t * QUART, QUART)],
                            out_l.at[pl.ds(hq, QUART)])
            plsc.subcore_barrier()

    rs = sid // TPS
    ts = sid % TPS
    rowS = cid * 4 + rs
    hqs = rowS * N + ts * QS

    def s1_chunk(c, acc):
        pltpu.sync_copy(la.at[pl.ds(hqs + c * CHUNK, CHUNK)], lbuf)

        def s1_col(j, a):
            return a + lbuf[pl.ds(j * 16, 16)]

        return lax.fori_loop(0, CK, s1_col, acc)

    acc16 = lax.fori_loop(0, NCHS, s1_chunk, zeros16)
    obuf[pl.ds(0, 16)] = acc16
    pltpu.sync_copy(obuf.at[pl.ds(0, 16)], ones_sh.at[rs, ts, pl.ds(0, 16)])
    plsc.subcore_barrier()
    for tt in range(TPS):
        pltpu.sync_copy(ones_sh.at[rs, tt, pl.ds(0, 16)],
                        obuf.at[pl.ds(tt * 16, 16)])
    qsums = [jnp.sum(obuf[pl.ds(tt * 16, 16)]) for tt in range(TPS)]
    g_tot = qsums[0] + qsums[1] + qsums[2] + qsums[3]
    cbase = jnp.int32(0)
    for tt in range(TPS - 1):
        cbase = cbase + jnp.where(ts > tt, qsums[tt], 0)
    gf = g_tot.astype(jnp.float32)

    def s2_chunk(c, carry):
        crun, accf = carry
        pltpu.sync_copy(ka.at[pl.ds(hqs + c * CHUNK, CHUNK)], kbuf)
        pltpu.sync_copy(la.at[pl.ds(hqs + c * CHUNK, CHUNK)], lbuf)

        def s2_col(j, jc):
            crun_j, af = jc
            key = kbuf[pl.ds(j * 16, 16)]
            lg = lbuf[pl.ds(j * 16, 16)]
            e = _key_to_err(key)
            r = jnp.maximum(e, 0.0)
            c_incl = crun_j + plsc.cumsum(lg)
            i1 = (ts * QS + c * CHUNK + j * 16 + 1) + ids
            z = i1 - c_incl
            u = gf + z.astype(jnp.float32)
            a = gf - c_incl.astype(jnp.float32)
            term1 = r / u
            term0 = jnp.where(u > 1.5, r * a / (u * (u - 1.0)), r)
            term = jnp.where(lg == 1, term1, term0)
            return crun_j + jnp.sum(lg), af + term

        return lax.fori_loop(0, CK, s2_col, (crun, accf))

    _, accf = lax.fori_loop(0, NCHS, s2_chunk,
                            (cbase, jnp.zeros((16,), jnp.float32)))
    obuf[pl.ds(16, 16)] = plsc.bitcast(accf, jnp.int32)
    pltpu.sync_copy(obuf.at[pl.ds(16, 16)], loss_hbm.at[pl.ds(wid * 16, 16)])


def _sc_call(x, lab):
    mesh = plsc.VectorSubcoreMesh(core_axis_name="c", subcore_axis_name="s")
    big = B * N
    f = pl.kernel(
        _sc_body,
        out_type=(jax.ShapeDtypeStruct((512,), jnp.int32),
                  jax.ShapeDtypeStruct((big,), jnp.int32),
                  jax.ShapeDtypeStruct((big,), jnp.int32),
                  jax.ShapeDtypeStruct((big,), jnp.int32),
                  jax.ShapeDtypeStruct((big,), jnp.int32)),
        mesh=mesh,
        scratch_types=[
            pltpu.VMEM((NB * 16,), jnp.int32),
            pltpu.VMEM((NB,), jnp.int32),
            pltpu.VMEM((NB,), jnp.int32),
            pltpu.VMEM((TPR * NB,), jnp.int32),
            pltpu.VMEM((CHUNK,), jnp.float32),
            pltpu.VMEM((CHUNK,), jnp.int32),
            pltpu.VMEM((CHUNK,), jnp.int32),
            pltpu.VMEM((CHUNK,), jnp.int32),
            pltpu.VMEM((64,), jnp.int32),
            pltpu.VMEM_SHARED((N,), jnp.int32),
            pltpu.VMEM_SHARED((N,), jnp.int32),
            pltpu.VMEM_SHARED((TPR, NB), jnp.int32),
            pltpu.VMEM_SHARED((4, TPS, 128), jnp.int32),
            pltpu.SemaphoreType.DMA,
        ],
        compiler_params=pltpu.CompilerParams(needs_layout_passes=False),
    )
    return f(x, lab)


@functools.partial(jax.jit, donate_argnums=())
def _run(x, lab):
    loss_bits = _sc_call(x, lab)[0]
    return jnp.sum(lax.bitcast_convert_type(loss_bits, jnp.float32)) / B


def kernel(input, target):
    x = input.reshape(-1)
    lab = target.reshape(-1).astype(jnp.int32)
    return _run(x, lab)


# speedup vs baseline: 1.0054x; 1.0054x over previous
import functools
import jax
import jax.numpy as jnp
from jax import lax
from jax.experimental import pallas as pl
from jax.experimental.pallas import tpu as pltpu
from jax.experimental.pallas import tpu_sc as plsc

B = 8
N = 262144
TPR = 16
QUART = N // TPR
LREG = QUART // 16
CK = 256
CHUNK = 16 * CK
NCH = LREG // CK
NB = 2048
SHIFTS = (0, 11, 22)
MASK = 0x7FF
TOPBIT = 0x7FFFFFFF
TPS = 4
QS = N // TPS
NCHS = QS // CHUNK


def _make_key(x, lab):
    signs = 2.0 * lab.astype(jnp.float32) - 1.0
    e = 1.0 - x * signs
    b = plsc.bitcast(e, jnp.int32)
    return jnp.where(b >= 0, b ^ TOPBIT, b)


def _key_to_err(key):
    b = jnp.where(key >= 0, key ^ TOPBIT, key)
    return plsc.bitcast(b, jnp.float32)


def _sc_body(x_hbm, lab_hbm, loss_hbm, ka, kb, la, lb,
             offh, dbase, ttot, bgrid, xbuf, kbuf, lbuf,
             istage, obuf, sk_sh, sl_sh, htot_sh, ones_sh, sem):
    cid = lax.axis_index("c")
    sid = lax.axis_index("s")
    wid = cid * 16 + sid
    t = sid
    ids = lax.iota(jnp.int32, 16)
    ones16 = jnp.ones((16,), jnp.int32)
    zeros16 = jnp.zeros((16,), jnp.int32)

    def clear_offh(_i, _c):
        for u in range(4):
            offh[pl.ds(_i * 64 + u * 16, 16)] = zeros16
        return 0

    def load_col(buf, j):
        return plsc.load_gather(buf, [ids * CK + jnp.broadcast_to(j, (16,))])

    for w in range(4):
        row = cid * 4 + w
        hq = row * N + t * QUART

        def stage_in(src, dst, c):
            descs = []
            for l in range(16):
                descs.append(pltpu.async_copy(
                    src.at[pl.ds(hq + l * LREG + c * CK, CK)],
                    dst.at[pl.ds(l * CK, CK)], sem))
            return descs

        for p in range(3):
            shift = SHIFTS[p]
            if p == 0:
                src_k, src_l = None, None
                out_k, out_l = ka, la
            elif p == 1:
                src_k, src_l = ka, la
                out_k, out_l = kb, lb
            else:
                src_k, src_l = kb, lb
                out_k, out_l = ka, la

            lax.fori_loop(0, NB * 16 // 64, clear_offh, 0)

            def a_chunk(c, _c):
                if p == 0:
                    d1 = stage_in(x_hbm, xbuf, c)
                    d2 = stage_in(lab_hbm, lbuf, c)
                    for d in d1 + d2:
                        d.wait()
                else:
                    for d in stage_in(src_k, kbuf, c):
                        d.wait()

                def a_col(j, _j):
                    for u in range(4):
                        jj = j * 4 + u
                        if p == 0:
                            key = _make_key(load_col(xbuf, jj),
                                            load_col(lbuf, jj))
                        else:
                            key = load_col(kbuf, jj)
                        dig = lax.shift_right_logical(key, shift) & MASK
                        plsc.addupdate_scatter(offh, [ids * NB + dig], ones16)
                    return 0

                lax.fori_loop(0, CK // 4, a_col, 0)
                return 0

            lax.fori_loop(0, NCH, a_chunk, 0)

            def b1(i, _c):
                acc = zeros16
                for l in range(16):
                    v = offh[pl.ds(l * NB + i * 16, 16)]
                    offh[pl.ds(l * NB + i * 16, 16)] = acc
                    acc = acc + v
                ttot[pl.ds(i * 16, 16)] = acc
                return 0

            lax.fori_loop(0, NB // 16, b1, 0)

            pltpu.sync_copy(ttot, htot_sh.at[t])
            plsc.subcore_barrier()
            for tt in range(TPR):
                pltpu.sync_copy(htot_sh.at[tt], bgrid.at[pl.ds(tt * NB, NB)])

            def b4(i, base):
                g = [bgrid[pl.ds(tt * NB + i * 16, 16)] for tt in range(TPR)]
                tsum = g[0]
                for tt in range(1, TPR):
                    tsum = tsum + g[tt]
                cs = plsc.cumsum(tsum)
                pre = zeros16
                for tt in range(TPR - 1):
                    pre = pre + jnp.where(t > tt, g[tt], zeros16)
                dbase[pl.ds(i * 16, 16)] = (cs - tsum) + pre + base
                return base + jnp.sum(tsum)

            lax.fori_loop(0, NB // 16, b4, jnp.int32(0))

            def c_chunk(c, _c):
                if p == 0:
                    d1 = stage_in(x_hbm, xbuf, c)
                    d2 = stage_in(lab_hbm, lbuf, c)
                else:
                    d1 = stage_in(src_k, kbuf, c)
                    d2 = stage_in(src_l, lbuf, c)
                for d in d1 + d2:
                    d.wait()

                def c_col(j, _j):
                    for u in range(2):
                        jj = j * 2 + u
                        jb = jnp.broadcast_to(jj, (16,))
                        lane_slot = ids * CK + jb
                        if p == 0:
                            lg = load_col(lbuf, jj)
                            key = _make_key(load_col(xbuf, jj), lg)
                            plsc.store_scatter(kbuf, [lane_slot], key)
                        else:
                            key = load_col(kbuf, jj)
                        dig = lax.shift_right_logical(key, shift) & MASK
                        p0 = plsc.load_gather(offh, [ids * NB + dig])
                        gb = plsc.load_gather(dbase, [dig])
                        plsc.store_scatter(offh, [ids * NB + dig], p0 + 1)
                        plsc.store_scatter(istage, [lane_slot], p0 + gb)
                    return 0

                lax.fori_loop(0, CK // 2, c_col, 0)
                pltpu.sync_copy(kbuf, sk_sh.at[istage])
                pltpu.sync_copy(lbuf, sl_sh.at[istage])
                return 0

            lax.fori_loop(0, NCH, c_chunk, 0)
            plsc.subcore_barrier()
            pltpu.sync_copy(sk_sh.at[pl.ds(t * QUART, QUART)],
                            out_k.at[pl.ds(hq, QUART)])
            pltpu.sync_copy(sl_sh.at[pl.ds(t * QUART, QUART)],
                            out_l.at[pl.ds(hq, QUART)])
            plsc.subcore_barrier()

    rs = sid // TPS
    ts = sid % TPS
    rowS = cid * 4 + rs
    hqs = rowS * N + ts * QS

    def s1_chunk(c, acc):
        pltpu.sync_copy(la.at[pl.ds(hqs + c * CHUNK, CHUNK)], lbuf)

        def s1_col(j, a):
            for u in range(4):
                a = a + lbuf[pl.ds((j * 4 + u) * 16, 16)]
            return a

        return lax.fori_loop(0, CK // 4, s1_col, acc)

    acc16 = lax.fori_loop(0, NCHS, s1_chunk, zeros16)
    obuf[pl.ds(0, 16)] = acc16
    pltpu.sync_copy(obuf.at[pl.ds(0, 16)], ones_sh.at[rs, ts, pl.ds(0, 16)])
    plsc.subcore_barrier()
    for tt in range(TPS):
        pltpu.sync_copy(ones_sh.at[rs, tt, pl.ds(0, 16)],
                        obuf.at[pl.ds(tt * 16, 16)])
    qsums = [jnp.sum(obuf[pl.ds(tt * 16, 16)]) for tt in range(TPS)]
    g_tot = qsums[0] + qsums[1] + qsums[2] + qsums[3]
    cbase = jnp.int32(0)
    for tt in range(TPS - 1):
        cbase = cbase + jnp.where(ts > tt, qsums[tt], 0)
    gf = g_tot.astype(jnp.float32)

    def s2_chunk(c, carry):
        crun, accf = carry
        pltpu.sync_copy(ka.at[pl.ds(hqs + c * CHUNK, CHUNK)], kbuf)
        pltpu.sync_copy(la.at[pl.ds(hqs + c * CHUNK, CHUNK)], lbuf)

        def s2_col(j, jc):
            crun_j, af = jc
            for v in range(2):
                jj = j * 2 + v
                key = kbuf[pl.ds(jj * 16, 16)]
                lg = lbuf[pl.ds(jj * 16, 16)]
                e = _key_to_err(key)
                r = jnp.maximum(e, 0.0)
                c_incl = crun_j + plsc.cumsum(lg)
                i1 = (ts * QS + c * CHUNK + jj * 16 + 1) + ids
                z = i1 - c_incl
                u = gf + z.astype(jnp.float32)
                a = gf - c_incl.astype(jnp.float32)
                term1 = r / u
                term0 = jnp.where(u > 1.5, r * a / (u * (u - 1.0)), r)
                term = jnp.where(lg == 1, term1, term0)
                crun_j = crun_j + jnp.sum(lg)
                af = af + term
            return crun_j, af

        return lax.fori_loop(0, CK // 2, s2_col, (crun, accf))

    _, accf = lax.fori_loop(0, NCHS, s2_chunk,
                            (cbase, jnp.zeros((16,), jnp.float32)))
    obuf[pl.ds(16, 16)] = plsc.bitcast(accf, jnp.int32)
    pltpu.sync_copy(obuf.at[pl.ds(16, 16)], loss_hbm.at[pl.ds(wid * 16, 16)])


def _sc_call(x, lab):
    mesh = plsc.VectorSubcoreMesh(core_axis_name="c", subcore_axis_name="s")
    big = B * N
    f = pl.kernel(
        _sc_body,
        out_type=(jax.ShapeDtypeStruct((512,), jnp.int32),
                  jax.ShapeDtypeStruct((big,), jnp.int32),
                  jax.ShapeDtypeStruct((big,), jnp.int32),
                  jax.ShapeDtypeStruct((big,), jnp.int32),
                  jax.ShapeDtypeStruct((big,), jnp.int32)),
        mesh=mesh,
        scratch_types=[
            pltpu.VMEM((NB * 16,), jnp.int32),
            pltpu.VMEM((NB,), jnp.int32),
            pltpu.VMEM((NB,), jnp.int32),
            pltpu.VMEM((TPR * NB,), jnp.int32),
            pltpu.VMEM((CHUNK,), jnp.float32),
            pltpu.VMEM((CHUNK,), jnp.int32),
            pltpu.VMEM((CHUNK,), jnp.int32),
            pltpu.VMEM((CHUNK,), jnp.int32),
            pltpu.VMEM((64,), jnp.int32),
            pltpu.VMEM_SHARED((N,), jnp.int32),
            pltpu.VMEM_SHARED((N,), jnp.int32),
            pltpu.VMEM_SHARED((TPR, NB), jnp.int32),
            pltpu.VMEM_SHARED((4, TPS, 128), jnp.int32),
            pltpu.SemaphoreType.DMA,
        ],
        compiler_params=pltpu.CompilerParams(needs_layout_passes=False),
    )
    return f(x, lab)


@functools.partial(jax.jit, donate_argnums=())
def _run(x, lab):
    loss_bits = _sc_call(x, lab)[0]
    return jnp.sum(lax.bitcast_convert_type(loss_bits, jnp.float32)) / B


def kernel(input, target):
    x = input.reshape(-1)
    lab = target.reshape(-1).astype(jnp.int32)
    return _run(x, lab)
